# baseline (speedup 1.0000x reference)
import functools

import jax
import jax.numpy as jnp
from jax import lax
from jax.experimental import pallas as pl
from jax.experimental.pallas import tpu as pltpu
from jax.experimental.pallas import tpu_sc as plsc

F32 = jnp.float32


def _mm_body(x_ref, w_ref, b_ref, o_ref):
    o_ref[...] = (
        jnp.dot(x_ref[...], w_ref[...], preferred_element_type=F32)
        + b_ref[...]
    )


def _mm(x, p, nblocks):
    n, din = x.shape
    dout = p["W"].shape[1]
    assert n % nblocks == 0
    blk = n // nblocks
    return pl.pallas_call(
        _mm_body,
        grid=(nblocks,),
        in_specs=[
            pl.BlockSpec((blk, din), lambda i: (i, 0)),
            pl.BlockSpec((din, dout), lambda i: (0, 0)),
            pl.BlockSpec((1, dout), lambda i: (0, 0)),
        ],
        out_specs=pl.BlockSpec((blk, dout), lambda i: (i, 0)),
        out_shape=jax.ShapeDtypeStruct((n, dout), F32),
    )(x, p["W"], p["b"].reshape(1, dout))


def _mm_ea_body(x_ref, w_ref, b_ref, o_ref):
    ea = (jnp.dot(x_ref[...], w_ref[...], preferred_element_type=F32)
          + b_ref[...])
    d = ea.shape[1]

    def f16bits(v):
        x = jax.lax.bitcast_convert_type(v, jnp.int32)
        rnd = x + 0xFFF + (jax.lax.shift_right_logical(x, 13) & 1)
        mag = rnd & 0x7FFFFFFF
        e = jax.lax.shift_right_logical(mag, 23)
        bits = jax.lax.shift_right_logical(mag, 13) - (112 << 10)
        bits = jnp.where(e > 112, bits, 0)
        sign = jax.lax.shift_right_logical(x, 16) & 0x8000
        return bits | sign

    lo = f16bits(ea[:, 0:d // 2])
    hi = f16bits(ea[:, d // 2:d])
    o_ref[...] = lo | jax.lax.shift_left(hi, 16)


def _pair_perm(dout):
    perm = [0] * dout
    for c in range(dout // 32):
        for k in range(16):
            perm[16 * c + k] = 32 * c + k
            perm[dout // 2 + 16 * c + k] = 32 * c + 16 + k
    return jnp.array(perm, jnp.int32)


def _mm_ea(x, p, nblocks):
    n, din = x.shape
    dout = p["W"].shape[1]
    blk = n // nblocks
    perm = _pair_perm(dout)
    return pl.pallas_call(
        _mm_ea_body,
        grid=(nblocks,),
        in_specs=[
            pl.BlockSpec((blk, din), lambda i: (i, 0)),
            pl.BlockSpec((din, dout), lambda i: (0, 0)),
            pl.BlockSpec((1, dout), lambda i: (0, 0)),
        ],
        out_specs=pl.BlockSpec((blk, dout // 2), lambda i: (i, 0)),
        out_shape=jax.ShapeDtypeStruct((n, dout // 2), jnp.int32),
    )(x, jnp.take(p["W"], perm, axis=1), jnp.take(p["b"], perm).reshape(1, dout))


BLK = 40
NB = 4


@functools.lru_cache(maxsize=None)
def _make_seg_mp(n_nodes, n_edges, d):
    info = plsc.get_sparse_core_info()
    nc, ns, lanes = info.num_cores, info.num_subcores, info.num_lanes
    nw = nc * ns
    assert n_edges % BLK == 0 and d % lanes == 0 and BLK % 8 == 0
    n_blocks = n_edges // BLK
    nfull = n_blocks // nw
    tail = n_blocks - nfull * nw
    assert nfull >= NB
    assert n_nodes % 8 == 0
    base = (n_nodes // (8 * ns)) * 8
    rem_groups = (n_nodes - base * ns) // 8
    chunks = []
    o = 0
    while o < base:
        chunks.append((o, min(BLK, base - o)))
        o += BLK

    def _when(cond, fn):
        if isinstance(cond, bool):
            if cond:
                fn()
        else:
            pl.when(cond)(fn)

    @functools.partial(
        pl.kernel,
        out_type=jax.ShapeDtypeStruct((nc, n_nodes, d), F32),
        mesh=plsc.VectorSubcoreMesh(core_axis_name="c", subcore_axis_name="s"),
        scratch_types=(
            [pltpu.VMEM_SHARED((n_nodes, d), F32)]
            + [pltpu.VMEM((BLK, d), F32)] * NB
            + [pltpu.VMEM((BLK, d // 2), jnp.int32)] * NB
            + [pltpu.VMEM((BLK,), jnp.int32)] * NB
            + [pltpu.VMEM((BLK,), jnp.int32)] * NB
            + [pltpu.SemaphoreType.DMA] * (5 * NB)
        ),
    )
    def seg_mp(table_hbm, ea_hbm, src_hbm, dst_hbm, out_hbm, accum, *bufs):
        gbufs = bufs[0:NB]
        ebufs = bufs[NB:2 * NB]
        svs = bufs[2 * NB:3 * NB]
        dvs = bufs[3 * NB:4 * NB]
        gsems = bufs[4 * NB:5 * NB]
        esems = bufs[5 * NB:6 * NB]
        ssems = bufs[6 * NB:7 * NB]
        isems = bufs[7 * NB:8 * NB]
        dsems = bufs[8 * NB:9 * NB]
        cid = lax.axis_index("c")
        sid = lax.axis_index("s")
        wid = sid * nc + cid
        msg_v = gbufs[0]

        def zrow(r, carry):
            for c in range(d // lanes):
                msg_v[r, pl.ds(c * lanes, lanes)] = jnp.zeros((lanes,), F32)
            return carry

        lax.fori_loop(0, BLK, zrow, 0)
        row0 = pl.multiple_of(sid * base, 8)
        for o, sz in chunks:
            pltpu.sync_copy(msg_v.at[pl.ds(0, sz)],
                            accum.at[pl.ds(row0 + o, sz)])
        for g in range(rem_groups):
            @pl.when(sid == g)
            def _():
                pltpu.sync_copy(msg_v.at[pl.ds(0, 8)],
                                accum.at[pl.ds(base * ns + g * 8, 8)])
        plsc.subcore_barrier()

        blk0 = wid * nfull

        def eoff(j):
            return pl.multiple_of((blk0 + j) * BLK, BLK)

        def issue_sidx(j, b):
            pltpu.async_copy(src_hbm.at[pl.ds(eoff(j), BLK)], svs[b], isems[b])

        def issue_didx(j, b):
            pltpu.async_copy(dst_hbm.at[pl.ds(eoff(j), BLK)], dvs[b], dsems[b])

        def issue_data(j, b):
            pltpu.async_copy(table_hbm.at[svs[b]], gbufs[b], gsems[b])
            pltpu.async_copy(ea_hbm.at[pl.ds(eoff(j), BLK)], ebufs[b],
                             esems[b])

        def drain_g(sem, ref):
            pltpu.make_async_copy(table_hbm.at[pl.ds(0, BLK)], ref, sem).wait()

        def drain_e(sem, ref):
            pltpu.make_async_copy(ea_hbm.at[pl.ds(0, BLK)], ref, sem).wait()

        def drain_idx(sem, ref):
            pltpu.make_async_copy(src_hbm.at[pl.ds(0, BLK)], ref, sem).wait()

        def valu(gb, eb):
            em_mask = jnp.full((lanes,), 0x7FFF0000, jnp.int32)
            sign_mask = jnp.full((lanes,), -2147483648, jnp.int32)
            hi_mask = jnp.full((lanes,), -65536, jnp.int32)
            bias = jnp.full((lanes,), 112 << 23, jnp.int32)
            zero = jnp.zeros((lanes,), jnp.int32)

            def half(x):
                em = jax.lax.shift_right_logical(x & em_mask, 3)
                f = (em + bias) | (x & sign_mask)
                f = jnp.where(em == zero, zero, f)
                return jax.lax.bitcast_convert_type(f, F32)

            @plsc.parallel_loop(0, BLK, step=1, unroll=4)
            def vrow(r):
                for c in range(d // 32):
                    v = eb[r, pl.ds(c * lanes, lanes)]
                    lo = half(jax.lax.shift_left(v, 16))
                    hi = half(v & hi_mask)
                    s0 = pl.ds(c * 32, lanes)
                    s1 = pl.ds(c * 32 + lanes, lanes)
                    gb[r, s0] = jnp.maximum(gb[r, s0] + lo, 0.0)
                    gb[r, s1] = jnp.maximum(gb[r, s1] + hi, 0.0)

        for j in range(NB):
            issue_sidx(j, j)
        for j in range(NB - 1):
            issue_didx(j, j)
        for j in range(NB - 1):
            drain_idx(isems[j], svs[j])
            issue_data(j, j)

        def step(j, b):
            b2 = (b + NB - 1) % NB
            drain_g(gsems[b], gbufs[b])
            drain_e(esems[b], ebufs[b])
            _when(_lt(j + NB, nfull), lambda: issue_sidx(j + NB, b))
            valu(gbufs[b], ebufs[b])
            drain_idx(dsems[b], dvs[b])
            pltpu.async_copy(gbufs[b], accum.at[dvs[b]], ssems[b], add=True)
            _when(_ge(j, 1), lambda: drain_g(ssems[b2], gbufs[b2]))
            def _prep():
                issue_didx(j + NB - 1, b2)
                drain_idx(isems[b2], svs[b2])
                issue_data(j + NB - 1, b2)
            _when(_lt(j + NB - 1, nfull), _prep)

        def _lt(a, lim):
            return (a < lim) if isinstance(a, int) else a < lim

        def _ge(a, lim):
            return (a >= lim) if isinstance(a, int) else a >= lim

        def ring_body(p, carry):
            for b in range(NB):
                step(p * NB + b, b)
            return carry

        lax.fori_loop(0, nfull // NB, ring_body, 0)
        for j in range(nfull - nfull % NB, nfull):
            step(j, j % NB)
        drain_g(ssems[(nfull - 1) % NB], gbufs[(nfull - 1) % NB])

        if tail:
            @pl.when(wid < tail)
            def _():
                tb = n_blocks - tail + wid
                off = pl.multiple_of(tb * BLK, BLK)
                pltpu.sync_copy(src_hbm.at[pl.ds(off, BLK)], svs[0])
                pltpu.sync_copy(dst_hbm.at[pl.ds(off, BLK)], dvs[0])
                cg = pltpu.async_copy(table_hbm.at[svs[0]], gbufs[0], gsems[0])
                ce = pltpu.async_copy(ea_hbm.at[pl.ds(off, BLK)], ebufs[0],
                                      esems[0])
                cg.wait()
                ce.wait()
                valu(gbufs[0], ebufs[0])
                pltpu.sync_copy(gbufs[0], accum.at[dvs[0]], add=True)

        plsc.subcore_barrier()

        for o, sz in chunks:
            pltpu.sync_copy(accum.at[pl.ds(row0 + o, sz)],
                            msg_v.at[pl.ds(0, sz)])
            pltpu.sync_copy(msg_v.at[pl.ds(0, sz)],
                            out_hbm.at[cid].at[pl.ds(row0 + o, sz)])
        for g in range(rem_groups):
            @pl.when(sid == g)
            def _():
                pltpu.sync_copy(accum.at[pl.ds(base * ns + g * 8, 8)],
                                msg_v.at[pl.ds(0, 8)])
                pltpu.sync_copy(msg_v.at[pl.ds(0, 8)],
                                out_hbm.at[cid].at[pl.ds(base * ns + g * 8, 8)])

    return seg_mp


def _fused_layer_body(n_nodes, relu_out, blk,
                      ax_ref, p0_ref, p1_ref, w1_ref, b1_ref, g1_ref, be1_ref,
                      w2_ref, b2_ref, g2_ref, be2_ref, o_ref, h1s, h2s, st):
    ph = pl.program_id(0)
    i = pl.program_id(1)
    rows = pl.ds(i * blk, blk)

    @pl.when(ph == 0)
    def _():
        h = ax_ref[...] + p0_ref[...] + p1_ref[...]
        h1 = jnp.dot(h, w1_ref[...], preferred_element_type=F32) + b1_ref[...]
        h1s[rows, :] = h1

        @pl.when(i == 0)
        def _():
            st[...] = jnp.zeros_like(st)

        st[0:1, :] = st[0:1, :] + jnp.sum(h1, axis=0, keepdims=True)
        st[1:2, :] = st[1:2, :] + jnp.sum(h1 * h1, axis=0, keepdims=True)

    @pl.when(ph == 1)
    def _():
        m = st[0:1, :] / n_nodes
        v = st[1:2, :] / n_nodes - m * m
        h1n = g1_ref[...] * (h1s[rows, :] - m) / jnp.sqrt(v + 1e-5)
        h1n = jnp.maximum(h1n + be1_ref[...], 0.0)
        h2 = jnp.dot(h1n, w2_ref[...], preferred_element_type=F32) + b2_ref[...]
        h2s[rows, :] = h2
        d = h2s.shape[1]

        @pl.when(i == 0)
        def _():
            st[2:3, 0:d] = jnp.zeros_like(st[2:3, 0:d])
            st[3:4, 0:d] = jnp.zeros_like(st[3:4, 0:d])

        st[2:3, 0:d] = st[2:3, 0:d] + jnp.sum(h2, axis=0, keepdims=True)
        st[3:4, 0:d] = st[3:4, 0:d] + jnp.sum(h2 * h2, axis=0, keepdims=True)

    @pl.when(ph == 2)
    def _():
        d = h2s.shape[1]
        m2 = st[2:3, 0:d] / n_nodes
        v2 = st[3:4, 0:d] / n_nodes - m2 * m2
        out = (g2_ref[...] * (h2s[rows, :] - m2) / jnp.sqrt(v2 + 1e-5)
               + be2_ref[...])
        if relu_out:
            out = jnp.maximum(out, 0.0)
        o_ref[...] = out


def _dense_layer(ax, parts, gp, bnp, relu, nblocks):
    n, d = ax.shape
    d2 = 2 * d
    assert n % nblocks == 0
    blk = n // nblocks
    row = lambda w: pl.BlockSpec((blk, w), lambda p, i: (i, 0))
    row0 = lambda w: pl.BlockSpec(
        (blk, w), lambda p, i: (jnp.where(p == 0, i, 0), 0))
    full = lambda a, w: pl.BlockSpec((a, w), lambda p, i: (0, 0))

    return pl.pallas_call(
        functools.partial(_fused_layer_body, float(n), relu, blk),
        grid=(3, nblocks),
        in_specs=[row0(d), row0(d), row0(d),
                  full(d, d2), full(1, d2), full(1, d2), full(1, d2),
                  full(d2, d), full(1, d), full(1, d), full(1, d)],
        out_specs=row(d),
        out_shape=jax.ShapeDtypeStruct((n, d), F32),
        scratch_shapes=[
            pltpu.VMEM((n, d2), F32),
            pltpu.VMEM((n, d), F32),
            pltpu.VMEM((8, d2), F32),
        ],
    )(ax, parts[0], parts[1],
      gp["lin1"]["W"], gp["lin1"]["b"].reshape(1, d2),
      gp["g"].reshape(1, d2), gp["beta"].reshape(1, d2),
      gp["lin2"]["W"], gp["lin2"]["b"].reshape(1, d),
      bnp["g"].reshape(1, d), bnp["beta"].reshape(1, d))


def _pool_body(nseg, x_ref, seg_ref, sum_ref, cnt_ref):
    i = pl.program_id(0)
    onehot = (seg_ref[...] == lax.broadcasted_iota(jnp.int32, (1, nseg), 1))
    onehot = onehot.astype(F32)
    ssum = lax.dot_general(onehot, x_ref[...], (((0,), (0,)), ((), ())),
                           preferred_element_type=F32)
    ones = jnp.ones_like(x_ref[...])
    scnt = lax.dot_general(onehot, ones, (((0,), (0,)), ((), ())),
                           preferred_element_type=F32)

    @pl.when(i == 0)
    def _():
        sum_ref[...] = jnp.zeros_like(sum_ref)
        cnt_ref[...] = jnp.zeros_like(cnt_ref)

    sum_ref[...] = sum_ref[...] + ssum
    cnt_ref[...] = cnt_ref[...] + scnt


def _pool(x, seg, nseg, nblocks):
    n, d = x.shape
    assert n % nblocks == 0
    blk = n // nblocks
    return pl.pallas_call(
        functools.partial(_pool_body, nseg),
        grid=(nblocks,),
        in_specs=[pl.BlockSpec((blk, d), lambda i: (i, 0)),
                  pl.BlockSpec((blk, 1), lambda i: (i, 0))],
        out_specs=[pl.BlockSpec((nseg, d), lambda i: (0, 0)),
                   pl.BlockSpec((nseg, d), lambda i: (0, 0))],
        out_shape=[jax.ShapeDtypeStruct((nseg, d), F32),
                   jax.ShapeDtypeStruct((nseg, d), F32)],
    )(x, seg.reshape(n, 1))


def _final_body(asum_ref, acnt_ref, fsum_ref, fcnt_ref,
                wf1_ref, bf1_ref, wf2_ref, bf2_ref,
                wo1_ref, bo1_ref, wo2_ref, bo2_ref, o_ref):
    atom_g = asum_ref[...] / jnp.maximum(acnt_ref[...], 1.0)
    fg_g = fsum_ref[...] / jnp.maximum(fcnt_ref[...], 1.0)
    comb = jnp.concatenate([atom_g, fg_g], axis=1)
    t = jnp.dot(comb, wf1_ref[...], preferred_element_type=F32) + bf1_ref[...]
    flow = jnp.dot(jnp.maximum(t, 0.0), wf2_ref[...],
                   preferred_element_type=F32) + bf2_ref[...]
    refined = fg_g + flow
    final = jnp.concatenate([refined, atom_g], axis=1)
    t2 = jnp.dot(final, wo1_ref[...], preferred_element_type=F32) + bo1_ref[...]
    o_ref[...] = jnp.dot(jnp.maximum(t2, 0.0), wo2_ref[...],
                         preferred_element_type=F32) + bo2_ref[...]


def _final(asum, acnt, fsum, fcnt, params):
    nb, d = asum.shape
    d2 = 2 * d
    p1, p2, p3, p4 = (params["ffg1"], params["ffg2"],
                      params["out1"], params["out2"])
    args = [asum, acnt, fsum, fcnt,
            p1["W"], p1["b"].reshape(1, -1), p2["W"], p2["b"].reshape(1, -1),
            p3["W"], p3["b"].reshape(1, -1), p4["W"], p4["b"].reshape(1, -1)]
    return pl.pallas_call(
        _final_body,
        out_shape=jax.ShapeDtypeStruct((nb, d2), F32),
    )(*args)


def kernel(x, edge_index, edge_attr, batch, fg_x, fg_edge_index, fg_edge_attr,
           fg_batch, atom2fg_index, seqEncoderTensor, params):
    d = params["atom_emb"]["W"].shape[1]
    n_atoms = x.shape[0]
    n_fg = fg_x.shape[0]
    nb = seqEncoderTensor.shape[0]

    ax = _mm(x, params["atom_emb"], nblocks=10)
    fx = _mm(fg_x, params["fg_emb"], nblocks=1)

    src_a, dst_a = edge_index[0], edge_index[1]
    seg_a = _make_seg_mp(n_atoms, edge_index.shape[1], d)
    for i in range(3):
        ea = _mm_ea(edge_attr, params["bond_emb"][i], nblocks=20)
        parts = seg_a(ax, ea, src_a, dst_a)
        ax = _dense_layer(ax, parts, params["atom_gin"][i],
                          params["atom_bn"][i], relu=(i != 2), nblocks=10)

    src_f, dst_f = fg_edge_index[0], fg_edge_index[1]
    seg_f = _make_seg_mp(n_fg, fg_edge_index.shape[1], d)
    for i in range(2):
        ea = _mm_ea(fg_edge_attr, params["fg_edge_emb"][i], nblocks=2)
        parts = seg_f(fx, ea, src_f, dst_f)
        fx = _dense_layer(fx, parts, params["fg_gin"][i],
                          params["fg_bn"][i], relu=(i != 1), nblocks=1)

    asum, acnt = _pool(ax, batch, nb, nblocks=10)
    fsum, fcnt = _pool(fx, fg_batch, nb, nblocks=1)
    return _final(asum, acnt, fsum, fcnt, params)


# speedup vs baseline: 1.0297x; 1.0297x over previous
import functools

import jax
import jax.numpy as jnp
from jax import lax
from jax.experimental import pallas as pl
from jax.experimental.pallas import tpu as pltpu
from jax.experimental.pallas import tpu_sc as plsc

F32 = jnp.float32


def _mm_body(x_ref, w_ref, b_ref, o_ref):
    o_ref[...] = (
        jnp.dot(x_ref[...], w_ref[...], preferred_element_type=F32)
        + b_ref[...]
    )


def _mm(x, p, nblocks):
    n, din = x.shape
    dout = p["W"].shape[1]
    assert n % nblocks == 0
    blk = n // nblocks
    return pl.pallas_call(
        _mm_body,
        grid=(nblocks,),
        in_specs=[
            pl.BlockSpec((blk, din), lambda i: (i, 0)),
            pl.BlockSpec((din, dout), lambda i: (0, 0)),
            pl.BlockSpec((1, dout), lambda i: (0, 0)),
        ],
        out_specs=pl.BlockSpec((blk, dout), lambda i: (i, 0)),
        out_shape=jax.ShapeDtypeStruct((n, dout), F32),
    )(x, p["W"], p["b"].reshape(1, dout))


def _mm_ea_body(x_ref, w_ref, b_ref, o_ref):
    ea = (jnp.dot(x_ref[...], w_ref[...], preferred_element_type=F32)
          + b_ref[...])
    d = ea.shape[1]

    def f16bits(v):
        x = jax.lax.bitcast_convert_type(v, jnp.int32)
        rnd = x + 0xFFF + (jax.lax.shift_right_logical(x, 13) & 1)
        mag = rnd & 0x7FFFFFFF
        e = jax.lax.shift_right_logical(mag, 23)
        bits = jax.lax.shift_right_logical(mag, 13) - (112 << 10)
        bits = jnp.where(e > 112, bits, 0)
        sign = jax.lax.shift_right_logical(x, 16) & 0x8000
        return bits | sign

    lo = f16bits(ea[:, 0:d // 2])
    hi = f16bits(ea[:, d // 2:d])
    o_ref[...] = lo | jax.lax.shift_left(hi, 16)


def _pair_perm(dout):
    perm = [0] * dout
    for c in range(dout // 32):
        for k in range(16):
            perm[16 * c + k] = 32 * c + k
            perm[dout // 2 + 16 * c + k] = 32 * c + 16 + k
    return jnp.array(perm, jnp.int32)


def _mm_ea(x, p, nblocks):
    n, din = x.shape
    dout = p["W"].shape[1]
    blk = n // nblocks
    perm = _pair_perm(dout)
    return pl.pallas_call(
        _mm_ea_body,
        grid=(nblocks,),
        in_specs=[
            pl.BlockSpec((blk, din), lambda i: (i, 0)),
            pl.BlockSpec((din, dout), lambda i: (0, 0)),
            pl.BlockSpec((1, dout), lambda i: (0, 0)),
        ],
        out_specs=pl.BlockSpec((blk, dout // 2), lambda i: (i, 0)),
        out_shape=jax.ShapeDtypeStruct((n, dout // 2), jnp.int32),
    )(x, jnp.take(p["W"], perm, axis=1), jnp.take(p["b"], perm).reshape(1, dout))


BLK = 40
NB = 4


@functools.lru_cache(maxsize=None)
def _make_seg_mp(n_nodes, n_edges, d):
    info = plsc.get_sparse_core_info()
    nc, ns, lanes = info.num_cores, info.num_subcores, info.num_lanes
    nw = nc * ns
    assert n_edges % BLK == 0 and d % lanes == 0 and BLK % 8 == 0
    n_blocks = n_edges // BLK
    nfull = n_blocks // nw
    tail = n_blocks - nfull * nw
    assert nfull >= NB
    assert n_nodes % 8 == 0
    base = (n_nodes // (8 * ns)) * 8
    rem_groups = (n_nodes - base * ns) // 8
    chunks = []
    o = 0
    while o < base:
        chunks.append((o, min(BLK, base - o)))
        o += BLK

    def _when(cond, fn):
        if isinstance(cond, bool):
            if cond:
                fn()
        else:
            pl.when(cond)(fn)

    @functools.partial(
        pl.kernel,
        out_type=jax.ShapeDtypeStruct((nc, n_nodes, d), F32),
        mesh=plsc.VectorSubcoreMesh(core_axis_name="c", subcore_axis_name="s"),
        scratch_types=(
            [pltpu.VMEM_SHARED((n_nodes, d), F32)]
            + [pltpu.VMEM((BLK, d), F32)] * NB
            + [pltpu.VMEM((BLK, d), F32)] * NB
            + [pltpu.VMEM((BLK,), jnp.int32)] * NB
            + [pltpu.VMEM((BLK,), jnp.int32)] * NB
            + [pltpu.SemaphoreType.DMA] * (5 * NB)
        ),
    )
    def seg_mp(table_hbm, ea_hbm, src_hbm, dst_hbm, out_hbm, accum, *bufs):
        gbufs = bufs[0:NB]
        ebufs = bufs[NB:2 * NB]
        svs = bufs[2 * NB:3 * NB]
        dvs = bufs[3 * NB:4 * NB]
        gsems = bufs[4 * NB:5 * NB]
        esems = bufs[5 * NB:6 * NB]
        ssems = bufs[6 * NB:7 * NB]
        isems = bufs[7 * NB:8 * NB]
        dsems = bufs[8 * NB:9 * NB]
        cid = lax.axis_index("c")
        sid = lax.axis_index("s")
        wid = sid * nc + cid
        msg_v = gbufs[0]

        def zrow(r, carry):
            for c in range(d // lanes):
                msg_v[r, pl.ds(c * lanes, lanes)] = jnp.zeros((lanes,), F32)
            return carry

        lax.fori_loop(0, BLK, zrow, 0)
        row0 = pl.multiple_of(sid * base, 8)
        for o, sz in chunks:
            pltpu.sync_copy(msg_v.at[pl.ds(0, sz)],
                            accum.at[pl.ds(row0 + o, sz)])
        for g in range(rem_groups):
            @pl.when(sid == g)
            def _():
                pltpu.sync_copy(msg_v.at[pl.ds(0, 8)],
                                accum.at[pl.ds(base * ns + g * 8, 8)])
        plsc.subcore_barrier()

        blk0 = wid * nfull

        def eoff(j):
            return pl.multiple_of((blk0 + j) * BLK, BLK)

        def issue_sidx(j, b):
            pltpu.async_copy(src_hbm.at[pl.ds(eoff(j), BLK)], svs[b], isems[b])

        def issue_didx(j, b):
            pltpu.async_copy(dst_hbm.at[pl.ds(eoff(j), BLK)], dvs[b], dsems[b])

        def issue_data(j, b):
            pltpu.async_copy(table_hbm.at[svs[b]], gbufs[b], gsems[b])
            pltpu.async_copy(ea_hbm.at[pl.ds(eoff(j), BLK)], ebufs[b],
                             esems[b])

        def drain_g(sem, ref):
            pltpu.make_async_copy(table_hbm.at[pl.ds(0, BLK)], ref, sem).wait()

        def drain_e(sem, ref):
            pltpu.make_async_copy(ea_hbm.at[pl.ds(0, BLK)], ref, sem).wait()

        def drain_idx(sem, ref):
            pltpu.make_async_copy(src_hbm.at[pl.ds(0, BLK)], ref, sem).wait()

        def valu(gb, eb):
            @plsc.parallel_loop(0, BLK, step=1, unroll=4)
            def vrow(r):
                for c in range(d // lanes):
                    s = pl.ds(c * lanes, lanes)
                    gb[r, s] = jnp.maximum(gb[r, s] + eb[r, s], 0.0)

        for j in range(NB):
            issue_sidx(j, j)
        for j in range(NB - 1):
            issue_didx(j, j)
        for j in range(NB - 1):
            drain_idx(isems[j], svs[j])
            issue_data(j, j)

        def step(j, b):
            b2 = (b + NB - 1) % NB
            drain_g(gsems[b], gbufs[b])
            drain_e(esems[b], ebufs[b])
            _when(_lt(j + NB, nfull), lambda: issue_sidx(j + NB, b))
            valu(gbufs[b], ebufs[b])
            drain_idx(dsems[b], dvs[b])
            pltpu.async_copy(gbufs[b], accum.at[dvs[b]], ssems[b], add=True)
            _when(_ge(j, 1), lambda: drain_g(ssems[b2], gbufs[b2]))
            def _prep():
                issue_didx(j + NB - 1, b2)
                drain_idx(isems[b2], svs[b2])
                issue_data(j + NB - 1, b2)
            _when(_lt(j + NB - 1, nfull), _prep)

        def _lt(a, lim):
            return (a < lim) if isinstance(a, int) else a < lim

        def _ge(a, lim):
            return (a >= lim) if isinstance(a, int) else a >= lim

        def ring_body(p, carry):
            for b in range(NB):
                step(p * NB + b, b)
            return carry

        lax.fori_loop(0, nfull // NB, ring_body, 0)
        for j in range(nfull - nfull % NB, nfull):
            step(j, j % NB)
        drain_g(ssems[(nfull - 1) % NB], gbufs[(nfull - 1) % NB])

        if tail:
            @pl.when(wid < tail)
            def _():
                tb = n_blocks - tail + wid
                off = pl.multiple_of(tb * BLK, BLK)
                pltpu.sync_copy(src_hbm.at[pl.ds(off, BLK)], svs[0])
                pltpu.sync_copy(dst_hbm.at[pl.ds(off, BLK)], dvs[0])
                cg = pltpu.async_copy(table_hbm.at[svs[0]], gbufs[0], gsems[0])
                ce = pltpu.async_copy(ea_hbm.at[pl.ds(off, BLK)], ebufs[0],
                                      esems[0])
                cg.wait()
                ce.wait()
                valu(gbufs[0], ebufs[0])
                pltpu.sync_copy(gbufs[0], accum.at[dvs[0]], add=True)

        plsc.subcore_barrier()

        pltpu.sync_copy(accum.at[pl.ds(row0, base)],
                        out_hbm.at[cid].at[pl.ds(row0, base)])
        for g in range(rem_groups):
            @pl.when(sid == g)
            def _():
                pltpu.sync_copy(accum.at[pl.ds(base * ns + g * 8, 8)],
                                out_hbm.at[cid].at[pl.ds(base * ns + g * 8, 8)])

    return seg_mp


def _fused_layer_body(n_nodes, relu_out, blk,
                      ax_ref, p0_ref, p1_ref, w1_ref, b1_ref, g1_ref, be1_ref,
                      w2_ref, b2_ref, g2_ref, be2_ref, o_ref, h1s, h2s, st):
    ph = pl.program_id(0)
    i = pl.program_id(1)
    rows = pl.ds(i * blk, blk)

    @pl.when(ph == 0)
    def _():
        h = ax_ref[...] + p0_ref[...] + p1_ref[...]
        h1 = jnp.dot(h, w1_ref[...], preferred_element_type=F32) + b1_ref[...]
        h1s[rows, :] = h1

        @pl.when(i == 0)
        def _():
            st[...] = jnp.zeros_like(st)

        st[0:1, :] = st[0:1, :] + jnp.sum(h1, axis=0, keepdims=True)
        st[1:2, :] = st[1:2, :] + jnp.sum(h1 * h1, axis=0, keepdims=True)

    @pl.when(ph == 1)
    def _():
        m = st[0:1, :] / n_nodes
        v = st[1:2, :] / n_nodes - m * m
        h1n = g1_ref[...] * (h1s[rows, :] - m) / jnp.sqrt(v + 1e-5)
        h1n = jnp.maximum(h1n + be1_ref[...], 0.0)
        h2 = jnp.dot(h1n, w2_ref[...], preferred_element_type=F32) + b2_ref[...]
        h2s[rows, :] = h2
        d = h2s.shape[1]

        @pl.when(i == 0)
        def _():
            st[2:3, 0:d] = jnp.zeros_like(st[2:3, 0:d])
            st[3:4, 0:d] = jnp.zeros_like(st[3:4, 0:d])

        st[2:3, 0:d] = st[2:3, 0:d] + jnp.sum(h2, axis=0, keepdims=True)
        st[3:4, 0:d] = st[3:4, 0:d] + jnp.sum(h2 * h2, axis=0, keepdims=True)

    @pl.when(ph == 2)
    def _():
        d = h2s.shape[1]
        m2 = st[2:3, 0:d] / n_nodes
        v2 = st[3:4, 0:d] / n_nodes - m2 * m2
        out = (g2_ref[...] * (h2s[rows, :] - m2) / jnp.sqrt(v2 + 1e-5)
               + be2_ref[...])
        if relu_out:
            out = jnp.maximum(out, 0.0)
        o_ref[...] = out


def _dense_layer(ax, parts, gp, bnp, relu, nblocks):
    n, d = ax.shape
    d2 = 2 * d
    assert n % nblocks == 0
    blk = n // nblocks
    row = lambda w: pl.BlockSpec((blk, w), lambda p, i: (i, 0))
    row0 = lambda w: pl.BlockSpec(
        (blk, w), lambda p, i: (jnp.where(p == 0, i, 0), 0))
    full = lambda a, w: pl.BlockSpec((a, w), lambda p, i: (0, 0))

    return pl.pallas_call(
        functools.partial(_fused_layer_body, float(n), relu, blk),
        grid=(3, nblocks),
        in_specs=[row0(d), row0(d), row0(d),
                  full(d, d2), full(1, d2), full(1, d2), full(1, d2),
                  full(d2, d), full(1, d), full(1, d), full(1, d)],
        out_specs=row(d),
        out_shape=jax.ShapeDtypeStruct((n, d), F32),
        scratch_shapes=[
            pltpu.VMEM((n, d2), F32),
            pltpu.VMEM((n, d), F32),
            pltpu.VMEM((8, d2), F32),
        ],
    )(ax, parts[0], parts[1],
      gp["lin1"]["W"], gp["lin1"]["b"].reshape(1, d2),
      gp["g"].reshape(1, d2), gp["beta"].reshape(1, d2),
      gp["lin2"]["W"], gp["lin2"]["b"].reshape(1, d),
      bnp["g"].reshape(1, d), bnp["beta"].reshape(1, d))


def _pool_body(nseg, x_ref, seg_ref, sum_ref, cnt_ref):
    i = pl.program_id(0)
    onehot = (seg_ref[...] == lax.broadcasted_iota(jnp.int32, (1, nseg), 1))
    onehot = onehot.astype(F32)
    ssum = lax.dot_general(onehot, x_ref[...], (((0,), (0,)), ((), ())),
                           preferred_element_type=F32)
    ones = jnp.ones_like(x_ref[...])
    scnt = lax.dot_general(onehot, ones, (((0,), (0,)), ((), ())),
                           preferred_element_type=F32)

    @pl.when(i == 0)
    def _():
        sum_ref[...] = jnp.zeros_like(sum_ref)
        cnt_ref[...] = jnp.zeros_like(cnt_ref)

    sum_ref[...] = sum_ref[...] + ssum
    cnt_ref[...] = cnt_ref[...] + scnt


def _pool(x, seg, nseg, nblocks):
    n, d = x.shape
    assert n % nblocks == 0
    blk = n // nblocks
    return pl.pallas_call(
        functools.partial(_pool_body, nseg),
        grid=(nblocks,),
        in_specs=[pl.BlockSpec((blk, d), lambda i: (i, 0)),
                  pl.BlockSpec((blk, 1), lambda i: (i, 0))],
        out_specs=[pl.BlockSpec((nseg, d), lambda i: (0, 0)),
                   pl.BlockSpec((nseg, d), lambda i: (0, 0))],
        out_shape=[jax.ShapeDtypeStruct((nseg, d), F32),
                   jax.ShapeDtypeStruct((nseg, d), F32)],
    )(x, seg.reshape(n, 1))


def _final_body(asum_ref, acnt_ref, fsum_ref, fcnt_ref,
                wf1_ref, bf1_ref, wf2_ref, bf2_ref,
                wo1_ref, bo1_ref, wo2_ref, bo2_ref, o_ref):
    atom_g = asum_ref[...] / jnp.maximum(acnt_ref[...], 1.0)
    fg_g = fsum_ref[...] / jnp.maximum(fcnt_ref[...], 1.0)
    comb = jnp.concatenate([atom_g, fg_g], axis=1)
    t = jnp.dot(comb, wf1_ref[...], preferred_element_type=F32) + bf1_ref[...]
    flow = jnp.dot(jnp.maximum(t, 0.0), wf2_ref[...],
                   preferred_element_type=F32) + bf2_ref[...]
    refined = fg_g + flow
    final = jnp.concatenate([refined, atom_g], axis=1)
    t2 = jnp.dot(final, wo1_ref[...], preferred_element_type=F32) + bo1_ref[...]
    o_ref[...] = jnp.dot(jnp.maximum(t2, 0.0), wo2_ref[...],
                         preferred_element_type=F32) + bo2_ref[...]


def _final(asum, acnt, fsum, fcnt, params):
    nb, d = asum.shape
    d2 = 2 * d
    p1, p2, p3, p4 = (params["ffg1"], params["ffg2"],
                      params["out1"], params["out2"])
    args = [asum, acnt, fsum, fcnt,
            p1["W"], p1["b"].reshape(1, -1), p2["W"], p2["b"].reshape(1, -1),
            p3["W"], p3["b"].reshape(1, -1), p4["W"], p4["b"].reshape(1, -1)]
    return pl.pallas_call(
        _final_body,
        out_shape=jax.ShapeDtypeStruct((nb, d2), F32),
    )(*args)


def kernel(x, edge_index, edge_attr, batch, fg_x, fg_edge_index, fg_edge_attr,
           fg_batch, atom2fg_index, seqEncoderTensor, params):
    d = params["atom_emb"]["W"].shape[1]
    n_atoms = x.shape[0]
    n_fg = fg_x.shape[0]
    nb = seqEncoderTensor.shape[0]

    ax = _mm(x, params["atom_emb"], nblocks=10)
    fx = _mm(fg_x, params["fg_emb"], nblocks=1)

    src_a, dst_a = edge_index[0], edge_index[1]
    seg_a = _make_seg_mp(n_atoms, edge_index.shape[1], d)
    for i in range(3):
        ea = _mm(edge_attr, params["bond_emb"][i], nblocks=20)
        parts = seg_a(ax, ea, src_a, dst_a)
        ax = _dense_layer(ax, parts, params["atom_gin"][i],
                          params["atom_bn"][i], relu=(i != 2), nblocks=10)

    src_f, dst_f = fg_edge_index[0], fg_edge_index[1]
    seg_f = _make_seg_mp(n_fg, fg_edge_index.shape[1], d)
    for i in range(2):
        ea = _mm(fg_edge_attr, params["fg_edge_emb"][i], nblocks=2)
        parts = seg_f(fx, ea, src_f, dst_f)
        fx = _dense_layer(fx, parts, params["fg_gin"][i],
                          params["fg_bn"][i], relu=(i != 1), nblocks=1)

    asum, acnt = _pool(ax, batch, nb, nblocks=10)
    fsum, fcnt = _pool(fx, fg_batch, nb, nblocks=1)
    return _final(asum, acnt, fsum, fcnt, params)
